# baseline (speedup 1.0000x reference)
import functools

import jax
import jax.numpy as jnp
from jax.experimental import pallas as pl
from jax.experimental.pallas import tpu as pltpu


def _body(facs_ref, x_ref, noise_ref, o_ref, pooled_ref, *, w, block, steps):
    core = pl.program_id(0)
    j = pl.program_id(1)

    @pl.when(j == 0)
    def _pool():
        v = x_ref[...].astype(jnp.float32)
        top = v[:, 0:w]
        bot = v[:, w:2 * w]
        rs = top + bot
        rm = jnp.maximum(top, bot)
        i = jax.lax.broadcasted_iota(jnp.int32, (w, w // 2), 0)
        jj = jax.lax.broadcasted_iota(jnp.int32, (w, w // 2), 1)
        e0 = (i == 2 * jj).astype(jnp.float32)
        e1 = (i == 2 * jj + 1).astype(jnp.float32)

        def dot(a, b):
            return jax.lax.dot_general(
                a, b, (((1,), (0,)), ((), ())),
                preferred_element_type=jnp.float32)

        def sel_dot(a, b):
            hi = a.astype(jnp.bfloat16).astype(jnp.float32)
            lo = a - hi
            return dot(hi, b) + dot(lo, b)

        cs_ = sel_dot(rs, e0 + e1)
        cm = jnp.maximum(sel_dot(rm, e0), sel_dot(rm, e1))
        pooled_ref[...] = ((cs_ * 0.25 + cm) * 0.5).reshape(pooled_ref.shape)

    pooled = pooled_ref[...]
    base = (core * steps + j) * block
    for b in range(block):
        fac = facs_ref[base + b]
        o_ref[b] = (pooled + fac * noise_ref[b].astype(jnp.float32)).astype(
            o_ref.dtype)


def kernel(x, facs, noise):
    N, C, H, W = x.shape
    cutn, _, cs, _ = noise.shape
    rows = C * cs
    x2 = x[0].reshape(rows, 2 * W)

    B = 2
    steps = cutn // (2 * B)
    out = pl.pallas_call(
        functools.partial(_body, w=W, block=B, steps=steps),
        out_shape=jax.ShapeDtypeStruct((cutn, C, cs, cs), x.dtype),
        grid=(2, steps),
        in_specs=[
            pl.BlockSpec(memory_space=pltpu.MemorySpace.SMEM),
            pl.BlockSpec((rows, 2 * W), lambda c, j: (0, 0)),
            pl.BlockSpec((B, C, cs, cs), lambda c, j: (c * steps + j, 0, 0, 0)),
        ],
        out_specs=pl.BlockSpec((B, C, cs, cs),
                               lambda c, j: (c * steps + j, 0, 0, 0)),
        scratch_shapes=[pltpu.VMEM((C, cs, cs), jnp.float32)],
        compiler_params=pltpu.CompilerParams(
            dimension_semantics=("parallel", "arbitrary"),
            vmem_limit_bytes=32 * 1024 * 1024,
        ),
    )(facs, x2, noise)

    return out


# speedup vs baseline: 1.2576x; 1.2576x over previous
import functools

import jax
import jax.numpy as jnp
from jax.experimental import pallas as pl
from jax.experimental.pallas import tpu as pltpu


def _body(facs_ref, x_ref, noise_ref, o_ref, pooled_ref, *, w, block, steps):
    core = pl.program_id(0)
    j = pl.program_id(1)

    @pl.when(j == 0)
    def _pool():
        v = x_ref[...].astype(jnp.float32)
        top = v[:, 0:w]
        bot = v[:, w:2 * w]
        rs = top + bot
        rm = jnp.maximum(top, bot)
        i = jax.lax.broadcasted_iota(jnp.int32, (w, w // 2), 0)
        jj = jax.lax.broadcasted_iota(jnp.int32, (w, w // 2), 1)
        e0 = (i == 2 * jj).astype(jnp.float32)
        e1 = (i == 2 * jj + 1).astype(jnp.float32)

        def dot(a, b):
            return jax.lax.dot_general(
                a, b, (((1,), (0,)), ((), ())),
                preferred_element_type=jnp.float32)

        def sel_dot(a, b):
            hi = a.astype(jnp.bfloat16).astype(jnp.float32)
            lo = a - hi
            return dot(hi, b) + dot(lo, b)

        cs_ = sel_dot(rs, e0 + e1)
        cm = jnp.maximum(sel_dot(rm, e0), sel_dot(rm, e1))
        pooled_ref[...] = ((cs_ * 0.25 + cm) * 0.5).reshape(pooled_ref.shape)

    pooled = pooled_ref[...]
    base = (core * steps + j) * block
    for b in range(block):
        fac = facs_ref[base + b]
        o_ref[b] = (pooled + fac * noise_ref[b].astype(jnp.float32)).astype(
            o_ref.dtype)


def kernel(x, facs, noise):
    N, C, H, W = x.shape
    cutn, _, cs, _ = noise.shape
    rows = C * cs
    x2 = x[0].reshape(rows, 2 * W)

    B = 8
    steps = cutn // (2 * B)
    out = pl.pallas_call(
        functools.partial(_body, w=W, block=B, steps=steps),
        out_shape=jax.ShapeDtypeStruct((cutn, C, cs, cs), x.dtype),
        grid=(2, steps),
        in_specs=[
            pl.BlockSpec(memory_space=pltpu.MemorySpace.SMEM),
            pl.BlockSpec((rows, 2 * W), lambda c, j: (0, 0)),
            pl.BlockSpec((B, C, cs, cs), lambda c, j: (c * steps + j, 0, 0, 0)),
        ],
        out_specs=pl.BlockSpec((B, C, cs, cs),
                               lambda c, j: (c * steps + j, 0, 0, 0)),
        scratch_shapes=[pltpu.VMEM((C, cs, cs), jnp.float32)],
        compiler_params=pltpu.CompilerParams(
            dimension_semantics=("parallel", "arbitrary"),
            vmem_limit_bytes=64 * 1024 * 1024,
        ),
    )(facs, x2, noise)

    return out


# speedup vs baseline: 1.3237x; 1.0525x over previous
import functools

import jax
import jax.numpy as jnp
from jax.experimental import pallas as pl
from jax.experimental.pallas import tpu as pltpu


def _body(facs_ref, x_ref, noise_ref, o_ref, pooled_ref, *, w, block, steps):
    core = pl.program_id(0)
    j = pl.program_id(1)

    @pl.when(j == 0)
    def _pool():
        v = x_ref[...].astype(jnp.float32)
        top = v[:, 0:w]
        bot = v[:, w:2 * w]
        rs = top + bot
        rm = jnp.maximum(top, bot)
        i = jax.lax.broadcasted_iota(jnp.int32, (w, w // 2), 0)
        jj = jax.lax.broadcasted_iota(jnp.int32, (w, w // 2), 1)
        e0 = (i == 2 * jj).astype(jnp.float32)
        e1 = (i == 2 * jj + 1).astype(jnp.float32)

        def dot(a, b):
            return jax.lax.dot_general(
                a, b, (((1,), (0,)), ((), ())),
                preferred_element_type=jnp.float32)

        def sel_dot(a, b):
            hi = a.astype(jnp.bfloat16).astype(jnp.float32)
            lo = a - hi
            return dot(hi, b) + dot(lo, b)

        cs_ = sel_dot(rs, e0 + e1)
        cm = jnp.maximum(sel_dot(rm, e0), sel_dot(rm, e1))
        pooled_ref[...] = ((cs_ * 0.25 + cm) * 0.5).reshape(pooled_ref.shape)

    pooled = pooled_ref[...]
    base = (core * steps + j) * block
    for b in range(block):
        fac = facs_ref[base + b]
        o_ref[b] = (pooled + fac * noise_ref[b].astype(jnp.float32)).astype(
            o_ref.dtype)


def kernel(x, facs, noise):
    N, C, H, W = x.shape
    cutn, _, cs, _ = noise.shape
    rows = C * cs
    x2 = x[0].reshape(rows, 2 * W)

    B = 16
    steps = cutn // (2 * B)
    out = pl.pallas_call(
        functools.partial(_body, w=W, block=B, steps=steps),
        out_shape=jax.ShapeDtypeStruct((cutn, C, cs, cs), x.dtype),
        grid=(2, steps),
        in_specs=[
            pl.BlockSpec(memory_space=pltpu.MemorySpace.SMEM),
            pl.BlockSpec((rows, 2 * W), lambda c, j: (0, 0)),
            pl.BlockSpec((B, C, cs, cs), lambda c, j: (c * steps + j, 0, 0, 0)),
        ],
        out_specs=pl.BlockSpec((B, C, cs, cs),
                               lambda c, j: (c * steps + j, 0, 0, 0)),
        scratch_shapes=[pltpu.VMEM((C, cs, cs), jnp.float32)],
        compiler_params=pltpu.CompilerParams(
            dimension_semantics=("parallel", "arbitrary"),
            vmem_limit_bytes=64 * 1024 * 1024,
        ),
    )(facs, x2, noise)

    return out


# speedup vs baseline: 1.3576x; 1.0256x over previous
import functools

import jax
import jax.numpy as jnp
from jax.experimental import pallas as pl
from jax.experimental.pallas import tpu as pltpu


def _body(facs_ref, x_ref, noise_ref, o_ref, *, w, cutn):
    c_dim, _, half, _ = x_ref.shape
    rows = c_dim * half
    v = x_ref[...].astype(jnp.float32).reshape(rows, 2 * w)
    top = v[:, 0:w]
    bot = v[:, w:2 * w]
    rs = top + bot
    rm = jnp.maximum(top, bot)
    i = jax.lax.broadcasted_iota(jnp.int32, (w, w // 2), 0)
    jj = jax.lax.broadcasted_iota(jnp.int32, (w, w // 2), 1)
    e0 = (i == 2 * jj).astype(jnp.float32)
    e1 = (i == 2 * jj + 1).astype(jnp.float32)

    def dot(a, b):
        return jax.lax.dot_general(
            a, b, (((1,), (0,)), ((), ())),
            preferred_element_type=jnp.float32)

    def sel_dot(a, b):
        hi = a.astype(jnp.bfloat16).astype(jnp.float32)
        lo = a - hi
        return dot(hi, b) + dot(lo, b)

    cs_ = sel_dot(rs, e0 + e1)
    cm = jnp.maximum(sel_dot(rm, e0), sel_dot(rm, e1))
    pooled = ((cs_ * 0.25 + cm) * 0.5).reshape(c_dim, half, w // 2)

    for b in range(cutn):
        fac = facs_ref[b]
        o_ref[b] = (pooled + fac * noise_ref[b].astype(jnp.float32)).astype(
            o_ref.dtype)


def kernel(x, facs, noise):
    N, C, H, W = x.shape
    cutn, _, cs, _ = noise.shape
    half = cs // 2
    x4 = x[0].reshape(C, 2, half, 2 * W)

    out = pl.pallas_call(
        functools.partial(_body, w=W, cutn=cutn),
        out_shape=jax.ShapeDtypeStruct((cutn, C, cs, cs), x.dtype),
        grid=(2,),
        in_specs=[
            pl.BlockSpec(memory_space=pltpu.MemorySpace.SMEM),
            pl.BlockSpec((C, 1, half, 2 * W), lambda h: (0, h, 0, 0)),
            pl.BlockSpec((cutn, C, half, cs), lambda h: (0, 0, h, 0)),
        ],
        out_specs=pl.BlockSpec((cutn, C, half, cs), lambda h: (0, 0, h, 0)),
        compiler_params=pltpu.CompilerParams(
            dimension_semantics=("parallel",),
            vmem_limit_bytes=64 * 1024 * 1024,
        ),
    )(facs, x4, noise)

    return out
